# baseline (device time: 33145 ns/iter reference)
import jax
import jax.numpy as jnp
from jax import lax
from jax.experimental import pallas as pl
from jax.experimental.pallas import tpu as pltpu


def kernel(O, Wo):
    B, S, H, D = O.shape
    K = H * D
    N = Wo.shape[1]
    S_half = S // 2

    O2 = jnp.swapaxes(O.reshape(B, S, K), 1, 2)

    def body(o_hbm, w_hbm, out_hbm, o_vmem, w_vmem, res_vmem,
             send_buf, recv_buf, load_sems, store_sems, send_sem, recv_sem):
        my_x = lax.axis_index("x")
        my_y = lax.axis_index("y")
        my_z = lax.axis_index("z")
        peer = (1 - my_x, my_y, my_z)

        w_load = pltpu.make_async_copy(w_hbm, w_vmem, load_sems.at[B])
        w_load.start()
        o_loads = []
        for b in range(B):
            cp = pltpu.make_async_copy(o_hbm.at[b], o_vmem.at[b], load_sems.at[b])
            cp.start()
            o_loads.append(cp)

        barrier = pltpu.get_barrier_semaphore()
        pl.semaphore_signal(
            barrier, inc=1, device_id=peer, device_id_type=pl.DeviceIdType.MESH
        )
        pl.semaphore_wait(barrier, 1)

        w_load.wait()
        w = w_vmem[...].astype(jnp.bfloat16)
        peer_start = (1 - my_x) * S_half
        my_start = my_x * S_half

        def matmul_shalf(b, start):
            o_b = o_vmem[b, :, pl.ds(start, S_half)].astype(jnp.bfloat16)
            return lax.dot_general(
                o_b, w,
                dimension_numbers=(((0,), (0,)), ((), ())),
                preferred_element_type=jnp.float32,
            )

        rdmas = []
        for b in range(B):
            o_loads[b].wait()
            send_buf[b] = matmul_shalf(b, peer_start).astype(jnp.bfloat16)
            rdma = pltpu.make_async_remote_copy(
                src_ref=send_buf.at[b],
                dst_ref=recv_buf.at[b],
                send_sem=send_sem.at[b],
                recv_sem=recv_sem.at[b],
                device_id=peer,
                device_id_type=pl.DeviceIdType.MESH,
            )
            rdma.start()
            rdmas.append(rdma)

        for b in range(B):
            res_vmem[b] = matmul_shalf(b, my_start)

        stores = []
        for b in range(B):
            rdmas[b].wait()
            res_vmem[b] = res_vmem[b] + recv_buf[b].astype(jnp.float32)
            st = pltpu.make_async_copy(
                res_vmem.at[b], out_hbm.at[b], store_sems.at[b]
            )
            st.start()
            stores.append(st)
        for st in stores:
            st.wait()

    return pl.pallas_call(
        body,
        out_shape=jax.ShapeDtypeStruct((B, S_half, N), jnp.float32),
        in_specs=[
            pl.BlockSpec(memory_space=pl.ANY),
            pl.BlockSpec(memory_space=pl.ANY),
        ],
        out_specs=pl.BlockSpec(memory_space=pl.ANY),
        scratch_shapes=[
            pltpu.VMEM((B, K, S), jnp.float32),
            pltpu.VMEM((K, N), jnp.float32),
            pltpu.VMEM((B, S_half, N), jnp.float32),
            pltpu.VMEM((B, S_half, N), jnp.bfloat16),
            pltpu.VMEM((B, S_half, N), jnp.bfloat16),
            pltpu.SemaphoreType.DMA((B + 1,)),
            pltpu.SemaphoreType.DMA((B,)),
            pltpu.SemaphoreType.DMA((B,)),
            pltpu.SemaphoreType.DMA((B,)),
        ],
        compiler_params=pltpu.CompilerParams(collective_id=0),
    )(O2, Wo)
